# baseline (speedup 1.0000x reference)
import functools

import jax
import jax.numpy as jnp
from jax import lax
from jax.experimental import pallas as pl
from jax.experimental.pallas import tpu as pltpu
from jax.experimental.pallas import tpu_sc as plsc

ALPHA = 0.1
K_STEPS = 2

NC = 2
NS = 16
CHUNK = 128
NBUF = 2
DNBUF = 4
DEGW = 16
FRAC0 = 1.0


def _deg_body(echunks_hbm, ones_hbm, zeros_hbm, deg_hbm,
              idxall_v, ones_v, tab_sh, *sems):
    cid = lax.axis_index("c")
    sid = lax.axis_index("s")
    _, _, nch, _ = echunks_hbm.shape
    N = tab_sh.shape[0]
    rows = N // NS
    row0 = sid * rows

    pltpu.sync_copy(ones_hbm, ones_v)
    pltpu.sync_copy(echunks_hbm.at[cid, sid], idxall_v)
    pltpu.sync_copy(zeros_hbm, tab_sh.at[pl.ds(row0, rows)])
    plsc.subcore_barrier()

    def body(g, carry):
        copies = []
        for b in range(DNBUF):
            c = g * DNBUF + b
            copies.append(pltpu.async_copy(
                ones_v, tab_sh.at[idxall_v.at[c]], sems[b], add=True))
        for cp in copies:
            cp.wait()
        return carry

    lax.fori_loop(0, nch // DNBUF, body, 0)
    plsc.subcore_barrier()

    sl = pl.ds(row0, rows)
    pltpu.sync_copy(tab_sh.at[sl], deg_hbm.at[cid].at[sl])


def _edge_body(c0, hs_hbm, echunks_hbm, zeros_hbm, agg_hbm,
               idxr_v, stage_v, agg_sh, *sems):
    cid = lax.axis_index("c")
    sid = lax.axis_index("s")
    nch_pad, _, _ = echunks_hbm.shape
    nch = nch_pad - NBUF
    nch0 = c0 // NS
    nch1 = (nch - c0) // NS
    base = jnp.where(cid == 0, sid * nch0, c0 + sid * nch1)
    ngroups = jnp.where(cid == 0, nch0 // NBUF, nch1 // NBUF)
    N, D = agg_sh.shape
    rows = N // NS
    row0 = sid * rows

    sem_i = sems[:2]
    sem_g = sems[2:2 + NBUF]
    sem_s = sems[2 + NBUF:]

    def idx_group(g):
        return echunks_hbm.at[pl.ds((base + g * NBUF) * 1, NBUF)]

    pltpu.async_copy(idx_group(0), idxr_v.at[0], sem_i[0])
    pltpu.sync_copy(zeros_hbm, agg_sh.at[pl.ds(row0, rows)])
    plsc.subcore_barrier()

    def body(g2, carry):
        for p in range(2):
            g = 2 * g2 + p
            pltpu.make_async_copy(idx_group(g), idxr_v.at[p], sem_i[p]).wait()
            pltpu.async_copy(idx_group(g + 1), idxr_v.at[1 - p], sem_i[1 - p])
            gathers = []
            for b in range(NBUF):
                gathers.append(pltpu.async_copy(
                    hs_hbm.at[idxr_v.at[p, b, 0]], stage_v.at[b], sem_g[b]))
            scatters = []
            for b in range(NBUF):
                gathers[b].wait()
                scatters.append(pltpu.async_copy(
                    stage_v.at[b], agg_sh.at[idxr_v.at[p, b, 1]], sem_s[b],
                    add=True))
            for cp in scatters:
                cp.wait()
        return carry

    lax.fori_loop(0, ngroups // 2, body, 0)
    pltpu.make_async_copy(idx_group(ngroups), idxr_v.at[0], sem_i[0]).wait()
    plsc.subcore_barrier()

    sl = pl.ds(row0, rows)
    pltpu.sync_copy(agg_sh.at[sl], agg_hbm.at[cid].at[sl])


def _linear_body(x_ref, w_ref, b_ref, od_ref, id_ref,
                 h0_ref, h0s_ref, ns_ref, nd_ref):
    h0 = jax.lax.dot_general(
        x_ref[...], w_ref[...], (((1,), (1,)), ((), ())),
        preferred_element_type=jnp.float32) + b_ref[...]
    ns = jax.lax.rsqrt(jnp.clip(od_ref[:, 0:1], 1.0, None))
    nd = jax.lax.rsqrt(jnp.clip(id_ref[:, 0:1], 1.0, None))
    h0_ref[...] = h0
    h0s_ref[...] = h0 * ns
    ns_ref[...] = jnp.broadcast_to(ns, ns_ref.shape)
    nd_ref[...] = jnp.broadcast_to(nd, nd_ref.shape)


def _blend_body(scale_src, use_b, *refs):
    if use_b:
        aggA_ref, aggB_ref, h0_ref, ns_ref, nd_ref, out_ref = refs
        agg = aggA_ref[...] + aggB_ref[...]
    else:
        aggA_ref, h0_ref, ns_ref, nd_ref, out_ref = refs
        agg = aggA_ref[...]
    h = (1.0 - ALPHA) * nd_ref[:, 0:1] * agg + ALPHA * h0_ref[...]
    if scale_src:
        h = h * ns_ref[:, 0:1]
    out_ref[...] = h


def kernel(x, edge_index, W, b):
    N0, D = x.shape
    E0 = edge_index.shape[1]

    f32 = jnp.float32
    mesh = plsc.VectorSubcoreMesh(core_axis_name="c", subcore_axis_name="s")

    quantum = NS * 8 * 10
    N = ((N0 + quantum - 1) // quantum) * quantum
    x = jnp.pad(x, ((0, N - N0), (0, 0)))

    ntiles = NC * NS
    equantum = ntiles * CHUNK * NBUF * 2
    assert (equantum // NC) % (CHUNK * DNBUF) == 0
    E = ((E0 + equantum - 1) // equantum) * equantum
    edges = jnp.pad(edge_index, ((0, 0), (0, E - E0)), constant_values=N - 1)

    C = E // CHUNK
    e_edge = edges.reshape(2, C, CHUNK).transpose(1, 0, 2)
    e_edge = jnp.concatenate(
        [e_edge, jnp.full((NBUF, 2, CHUNK), N - 1, jnp.int32)], axis=0)
    cquantum = NS * NBUF * 2
    c0_chunks = min(C, max(cquantum, int(round(FRAC0 * C / cquantum)) * cquantum))
    assert 0 < c0_chunks <= C and (C - c0_chunks) % cquantum == 0
    e_deg = edges.reshape(2, NS, E // (NS * CHUNK), CHUNK)
    nch_d = e_deg.shape[2]

    rows = N // NS
    ones_rows = jnp.zeros((CHUNK, D), f32).at[:, 0].set(1.0)
    zeros_agg = jnp.zeros((rows, D), f32)

    deg_kernel = pl.kernel(
        _deg_body,
        out_type=jax.ShapeDtypeStruct((NC, N, D), f32),
        mesh=mesh,
        scratch_types=[
            pltpu.VMEM((nch_d, CHUNK), jnp.int32),
            pltpu.VMEM((CHUNK, D), f32),
            pltpu.VMEM_SHARED((N, D), f32),
        ] + [pltpu.SemaphoreType.DMA] * DNBUF,
    )
    deg2 = deg_kernel(e_deg, ones_rows, zeros_agg)
    outdeg, indeg = deg2[0], deg2[1]

    grid = 10
    blk = N // grid
    linear = pl.pallas_call(
        _linear_body,
        grid=(grid,),
        in_specs=[
            pl.BlockSpec((blk, D), lambda i: (i, 0)),
            pl.BlockSpec((D, D), lambda i: (0, 0)),
            pl.BlockSpec((1, D), lambda i: (0, 0)),
            pl.BlockSpec((blk, D), lambda i: (i, 0)),
            pl.BlockSpec((blk, D), lambda i: (i, 0)),
        ],
        out_specs=[
            pl.BlockSpec((blk, D), lambda i: (i, 0)),
            pl.BlockSpec((blk, D), lambda i: (i, 0)),
            pl.BlockSpec((blk, DEGW), lambda i: (i, 0)),
            pl.BlockSpec((blk, DEGW), lambda i: (i, 0)),
        ],
        out_shape=[
            jax.ShapeDtypeStruct((N, D), f32),
            jax.ShapeDtypeStruct((N, D), f32),
            jax.ShapeDtypeStruct((N, DEGW), f32),
            jax.ShapeDtypeStruct((N, DEGW), f32),
        ],
    )
    h0, h0s, ns, nd = linear(x, W, b.reshape(1, D), outdeg, indeg)

    edge_kernel = pl.kernel(
        functools.partial(_edge_body, c0_chunks),
        out_type=jax.ShapeDtypeStruct((NC, N, D), f32),
        mesh=mesh,
        scratch_types=[
            pltpu.VMEM((2, NBUF, 2, CHUNK), jnp.int32),
            pltpu.VMEM((NBUF, CHUNK, D), f32),
            pltpu.VMEM_SHARED((N, D), f32),
        ] + [pltpu.SemaphoreType.DMA] * (2 + 2 * NBUF),
    )

    use_b = c0_chunks < C

    def blend(scale_src, aggs):
        nagg = len(aggs)
        return pl.pallas_call(
            functools.partial(_blend_body, scale_src, use_b),
            grid=(grid,),
            in_specs=[pl.BlockSpec((blk, D), lambda i: (i, 0))] * (nagg + 1) + [
                pl.BlockSpec((blk, DEGW), lambda i: (i, 0)),
                pl.BlockSpec((blk, DEGW), lambda i: (i, 0)),
            ],
            out_specs=pl.BlockSpec((blk, D), lambda i: (i, 0)),
            out_shape=jax.ShapeDtypeStruct((N, D), f32),
        )(*aggs, h0, ns, nd)

    h = h0s
    for step in range(K_STEPS):
        agg2 = edge_kernel(h, e_edge, zeros_agg)
        aggs = [agg2[0], agg2[1]] if use_b else [agg2[0]]
        h = blend(step < K_STEPS - 1, aggs)
    return h[:N0]


# speedup vs baseline: 1.4276x; 1.4276x over previous
import functools

import jax
import jax.numpy as jnp
from jax import lax
from jax.experimental import pallas as pl
from jax.experimental.pallas import tpu as pltpu
from jax.experimental.pallas import tpu_sc as plsc

ALPHA = 0.1
K_STEPS = 2

NC = 2
NS = 16
CHUNK = 128
NBUF = 2
DNBUF = 8
DEGW = 16
FRAC0 = 0.875


def _deg_body(echunks_hbm, ones_hbm, zeros_hbm, deg_hbm,
              idxall_v, ones_v, tab_sh, *sems):
    cid = lax.axis_index("c")
    sid = lax.axis_index("s")
    _, _, nch, _ = echunks_hbm.shape
    N = tab_sh.shape[0]
    rows = N // NS
    row0 = sid * rows

    pltpu.sync_copy(ones_hbm, ones_v)
    pltpu.sync_copy(echunks_hbm.at[cid, sid], idxall_v)
    pltpu.sync_copy(zeros_hbm, tab_sh.at[pl.ds(row0, rows)])
    plsc.subcore_barrier()

    def body(g, carry):
        copies = []
        for b in range(DNBUF):
            c = g * DNBUF + b
            copies.append(pltpu.async_copy(
                ones_v, tab_sh.at[idxall_v.at[c]], sems[b], add=True))
        for cp in copies:
            cp.wait()
        return carry

    lax.fori_loop(0, nch // DNBUF, body, 0)
    plsc.subcore_barrier()

    sl = pl.ds(row0, rows)
    pltpu.sync_copy(tab_sh.at[sl], deg_hbm.at[cid].at[sl])


def _edge_body(c0, hs_hbm, echunks_hbm, zeros_hbm, agg_hbm,
               idxr_v, stage_v, agg_sh, *sems):
    cid = lax.axis_index("c")
    sid = lax.axis_index("s")
    nch_pad, _, _ = echunks_hbm.shape
    nch = nch_pad - NBUF
    nch0 = c0 // NS
    nch1 = (nch - c0) // NS
    base = jnp.where(cid == 0, sid * nch0, c0 + sid * nch1)
    ngroups = jnp.where(cid == 0, nch0 // NBUF, nch1 // NBUF)
    N, D = agg_sh.shape
    rows = N // NS
    row0 = sid * rows

    sem_i = sems[:2]
    sem_g = sems[2:2 + NBUF]
    sem_s = sems[2 + NBUF:]

    def idx_group(g):
        return echunks_hbm.at[pl.ds((base + g * NBUF) * 1, NBUF)]

    pltpu.async_copy(idx_group(0), idxr_v.at[0], sem_i[0])
    pltpu.sync_copy(zeros_hbm, agg_sh.at[pl.ds(row0, rows)])
    plsc.subcore_barrier()

    def body(g2, carry):
        for p in range(2):
            g = 2 * g2 + p
            pltpu.make_async_copy(idx_group(g), idxr_v.at[p], sem_i[p]).wait()
            pltpu.async_copy(idx_group(g + 1), idxr_v.at[1 - p], sem_i[1 - p])
            gathers = []
            for b in range(NBUF):
                gathers.append(pltpu.async_copy(
                    hs_hbm.at[idxr_v.at[p, b, 0]], stage_v.at[b], sem_g[b]))
            scatters = []
            for b in range(NBUF):
                gathers[b].wait()
                scatters.append(pltpu.async_copy(
                    stage_v.at[b], agg_sh.at[idxr_v.at[p, b, 1]], sem_s[b],
                    add=True))
            for cp in scatters:
                cp.wait()
        return carry

    lax.fori_loop(0, ngroups // 2, body, 0)
    pltpu.make_async_copy(idx_group(ngroups), idxr_v.at[0], sem_i[0]).wait()
    plsc.subcore_barrier()

    sl = pl.ds(row0, rows)
    pltpu.sync_copy(agg_sh.at[sl], agg_hbm.at[cid].at[sl])


def _linear_body(x_ref, w_ref, b_ref, od_ref, id_ref,
                 h0_ref, h0s_ref, ns_ref, nd_ref):
    h0 = jax.lax.dot_general(
        x_ref[...], w_ref[...], (((1,), (1,)), ((), ())),
        preferred_element_type=jnp.float32) + b_ref[...]
    ns = jax.lax.rsqrt(jnp.clip(od_ref[:, 0:1], 1.0, None))
    nd = jax.lax.rsqrt(jnp.clip(id_ref[:, 0:1], 1.0, None))
    h0_ref[...] = h0
    h0s_ref[...] = h0 * ns
    ns_ref[...] = jnp.broadcast_to(ns, ns_ref.shape)
    nd_ref[...] = jnp.broadcast_to(nd, nd_ref.shape)


def _blend_body(scale_src, use_b, *refs):
    if use_b:
        aggA_ref, aggB_ref, h0_ref, ns_ref, nd_ref, out_ref = refs
        agg = aggA_ref[...] + aggB_ref[...]
    else:
        aggA_ref, h0_ref, ns_ref, nd_ref, out_ref = refs
        agg = aggA_ref[...]
    h = (1.0 - ALPHA) * nd_ref[:, 0:1] * agg + ALPHA * h0_ref[...]
    if scale_src:
        h = h * ns_ref[:, 0:1]
    out_ref[...] = h


def kernel(x, edge_index, W, b):
    N0, D = x.shape
    E0 = edge_index.shape[1]

    f32 = jnp.float32
    mesh = plsc.VectorSubcoreMesh(core_axis_name="c", subcore_axis_name="s")

    quantum = NS * 8 * 10
    N = ((N0 + quantum - 1) // quantum) * quantum
    x = jnp.pad(x, ((0, N - N0), (0, 0)))

    ntiles = NC * NS
    equantum = ntiles * CHUNK * NBUF * 2
    assert (equantum // NC) % (CHUNK * DNBUF) == 0
    E = ((E0 + equantum - 1) // equantum) * equantum
    edges = jnp.pad(edge_index, ((0, 0), (0, E - E0)), constant_values=N - 1)

    C = E // CHUNK
    e_edge = edges.reshape(2, C, CHUNK).transpose(1, 0, 2)
    e_edge = jnp.concatenate(
        [e_edge, jnp.full((NBUF, 2, CHUNK), N - 1, jnp.int32)], axis=0)
    cquantum = NS * NBUF * 2
    c0_chunks = min(C, max(cquantum, int(round(FRAC0 * C / cquantum)) * cquantum))
    assert 0 < c0_chunks <= C and (C - c0_chunks) % cquantum == 0
    e_deg = edges.reshape(2, NS, E // (NS * CHUNK), CHUNK)
    nch_d = e_deg.shape[2]

    rows = N // NS
    ones_rows = jnp.zeros((CHUNK, D), f32).at[:, 0].set(1.0)
    zeros_agg = jnp.zeros((rows, D), f32)

    deg_kernel = pl.kernel(
        _deg_body,
        out_type=jax.ShapeDtypeStruct((NC, N, D), f32),
        mesh=mesh,
        scratch_types=[
            pltpu.VMEM((nch_d, CHUNK), jnp.int32),
            pltpu.VMEM((CHUNK, D), f32),
            pltpu.VMEM_SHARED((N, D), f32),
        ] + [pltpu.SemaphoreType.DMA] * DNBUF,
    )
    deg2 = deg_kernel(e_deg, ones_rows, zeros_agg)
    outdeg, indeg = deg2[0], deg2[1]

    grid = 10
    blk = N // grid
    linear = pl.pallas_call(
        _linear_body,
        grid=(grid,),
        in_specs=[
            pl.BlockSpec((blk, D), lambda i: (i, 0)),
            pl.BlockSpec((D, D), lambda i: (0, 0)),
            pl.BlockSpec((1, D), lambda i: (0, 0)),
            pl.BlockSpec((blk, D), lambda i: (i, 0)),
            pl.BlockSpec((blk, D), lambda i: (i, 0)),
        ],
        out_specs=[
            pl.BlockSpec((blk, D), lambda i: (i, 0)),
            pl.BlockSpec((blk, D), lambda i: (i, 0)),
            pl.BlockSpec((blk, DEGW), lambda i: (i, 0)),
            pl.BlockSpec((blk, DEGW), lambda i: (i, 0)),
        ],
        out_shape=[
            jax.ShapeDtypeStruct((N, D), f32),
            jax.ShapeDtypeStruct((N, D), f32),
            jax.ShapeDtypeStruct((N, DEGW), f32),
            jax.ShapeDtypeStruct((N, DEGW), f32),
        ],
    )
    h0, h0s, ns, nd = linear(x, W, b.reshape(1, D), outdeg, indeg)

    edge_kernel = pl.kernel(
        functools.partial(_edge_body, c0_chunks),
        out_type=jax.ShapeDtypeStruct((NC, N, D), f32),
        mesh=mesh,
        scratch_types=[
            pltpu.VMEM((2, NBUF, 2, CHUNK), jnp.int32),
            pltpu.VMEM((NBUF, CHUNK, D), f32),
            pltpu.VMEM_SHARED((N, D), f32),
        ] + [pltpu.SemaphoreType.DMA] * (2 + 2 * NBUF),
    )

    use_b = c0_chunks < C

    def blend(scale_src, aggs):
        nagg = len(aggs)
        return pl.pallas_call(
            functools.partial(_blend_body, scale_src, use_b),
            grid=(grid,),
            in_specs=[pl.BlockSpec((blk, D), lambda i: (i, 0))] * (nagg + 1) + [
                pl.BlockSpec((blk, DEGW), lambda i: (i, 0)),
                pl.BlockSpec((blk, DEGW), lambda i: (i, 0)),
            ],
            out_specs=pl.BlockSpec((blk, D), lambda i: (i, 0)),
            out_shape=jax.ShapeDtypeStruct((N, D), f32),
        )(*aggs, h0, ns, nd)

    h = h0s
    for step in range(K_STEPS):
        agg2 = edge_kernel(h, e_edge, zeros_agg)
        aggs = [agg2[0], agg2[1]] if use_b else [agg2[0]]
        h = blend(step < K_STEPS - 1, aggs)
    return h[:N0]
